# baseline (speedup 1.0000x reference)
import jax
import jax.numpy as jnp
from jax import lax
from jax.experimental import pallas as pl
from jax.experimental.pallas import tpu as pltpu
from jax.experimental.pallas import tpu_sc as plsc

_N = 10000
_E = 320000
_D = 128
_B = 64
_DOUT = 1317
_DPAD = 1408
_NC, _NS = 2, 16
_NW = _NC * _NS
_EPT = _E // _NW
_CH = 128
_NCHUNK = 80
_EPAD = _NCHUNK * _CH
_NP = 10240
_STRIPE = _NP // _NS
_DUMMY = _N
_ROWBLK = 2000
_GRID = _N // _ROWBLK
_DT = jnp.float32


def _sc_deg_body(dst_hbm, ones_hbm, zeros_hbm, out_hbm, idx_v, ones_v, acc,
                 sem):
    c = lax.axis_index("c")
    s = lax.axis_index("s")
    wid = c * _NS + s
    pltpu.sync_copy(zeros_hbm, acc.at[pl.ds(s * _STRIPE, _STRIPE)])
    pltpu.sync_copy(ones_hbm, ones_v)
    pltpu.sync_copy(dst_hbm.at[wid], idx_v)
    plsc.subcore_barrier()

    def fire(j, carry):
        pltpu.async_copy(ones_v, acc.at[idx_v.at[j]], sem, add=True)
        return carry

    lax.fori_loop(0, _NCHUNK, fire, 0)

    def drain(j, carry):
        pltpu.make_async_copy(ones_hbm, ones_v, sem).wait()
        return carry

    lax.fori_loop(0, _NCHUNK, drain, 0)
    plsc.subcore_barrier()
    pltpu.sync_copy(acc.at[pl.ds(s * _STRIPE, _STRIPE)],
                    out_hbm.at[c, pl.ds(s * _STRIPE, _STRIPE)])


_sc_deg = pl.kernel(
    _sc_deg_body,
    out_type=jax.ShapeDtypeStruct((_NC, _NP, _D), _DT),
    mesh=plsc.VectorSubcoreMesh(core_axis_name="c", subcore_axis_name="s"),
    scratch_types=[
        pltpu.VMEM((_NCHUNK, _CH), jnp.int32),
        pltpu.VMEM((_CH, _D), _DT),
        pltpu.VMEM_SHARED((_NP, _D), _DT),
        pltpu.SemaphoreType.DMA,
    ],
)


def _sc_agg_body(hs_hbm, src_hbm, dst_hbm, zeros_hbm, out_hbm,
                 src_v, dst_v, rows_v, acc, sem):
    c = lax.axis_index("c")
    s = lax.axis_index("s")
    wid = c * _NS + s
    pltpu.sync_copy(zeros_hbm, acc.at[pl.ds(s * _STRIPE, _STRIPE)])
    pltpu.sync_copy(src_hbm.at[wid], src_v)
    pltpu.sync_copy(dst_hbm.at[wid], dst_v)
    plsc.subcore_barrier()

    def body(j, carry):
        pltpu.async_copy(hs_hbm.at[src_v.at[j]], rows_v, sem).wait()
        pltpu.sync_copy(rows_v, acc.at[dst_v.at[j]], add=True)
        return carry

    lax.fori_loop(0, _NCHUNK, body, 0)
    plsc.subcore_barrier()
    pltpu.sync_copy(acc.at[pl.ds(s * _STRIPE, _STRIPE)],
                    out_hbm.at[c, pl.ds(s * _STRIPE, _STRIPE)])


_sc_agg = pl.kernel(
    _sc_agg_body,
    out_type=jax.ShapeDtypeStruct((_NC, _NP, _D), _DT),
    mesh=plsc.VectorSubcoreMesh(core_axis_name="c", subcore_axis_name="s"),
    scratch_types=[
        pltpu.VMEM((_NCHUNK, _CH), jnp.int32),
        pltpu.VMEM((_NCHUNK, _CH), jnp.int32),
        pltpu.VMEM((_CH, _D), _DT),
        pltpu.VMEM_SHARED((_NP, _D), _DT),
        pltpu.SemaphoreType.DMA,
    ],
)


def _dinv(deg_ref):
    d = (deg_ref[0, :, 0:1].astype(jnp.float32) +
         deg_ref[1, :, 0:1].astype(jnp.float32))
    return lax.rsqrt(d + 1.0)


def _mm_first_body(x_ref, w_ref, deg_ref, o_ref):
    h = jnp.dot(x_ref[...], w_ref[...],
                preferred_element_type=jnp.float32) * _dinv(deg_ref)
    o_ref[...] = h.astype(_DT)


def _mm_mid_body(acc_ref, hs_ref, deg_ref, b_ref, w_ref, o_ref):
    dinv = _dinv(deg_ref)
    t = (acc_ref[0].astype(jnp.float32) + acc_ref[1].astype(jnp.float32) +
         hs_ref[...].astype(jnp.float32))
    xl = jnp.maximum(t * dinv + b_ref[...], 0.0)
    o_ref[...] = (jnp.dot(xl, w_ref[...],
                          preferred_element_type=jnp.float32) *
                  dinv).astype(_DT)


def _pool_body(acc_ref, hs_ref, deg_ref, b_ref, batch_ref, wf_ref, bf_ref,
               o_ref, sums, cnts):
    i = pl.program_id(0)
    dinv = _dinv(deg_ref)
    t = (acc_ref[0].astype(jnp.float32) + acc_ref[1].astype(jnp.float32) +
         hs_ref[...].astype(jnp.float32))
    xl = jnp.maximum(t * dinv + b_ref[...], 0.0)
    bids = batch_ref[0, 0, :]
    oh = (bids[None, :] ==
          lax.broadcasted_iota(jnp.int32, (_B, _ROWBLK), 0)).astype(jnp.float32)

    @pl.when(i == 0)
    def _():
        sums[...] = jnp.zeros_like(sums)
        cnts[...] = jnp.zeros_like(cnts)

    sums[...] += jnp.dot(oh, xl, preferred_element_type=jnp.float32)
    cnts[...] += jnp.broadcast_to(jnp.sum(oh, axis=1, keepdims=True),
                                  (_B, _D))

    @pl.when(i == _GRID - 1)
    def _():
        pooled = sums[...] / jnp.maximum(cnts[...], 1.0)
        logits = jnp.dot(pooled, wf_ref[...],
                         preferred_element_type=jnp.float32) + bf_ref[...]
        m = jnp.max(logits, axis=1, keepdims=True)
        lse = jnp.log(jnp.sum(jnp.exp(logits - m), axis=1, keepdims=True))
        o_ref[...] = logits - m - lse


_row_spec = pl.BlockSpec((_ROWBLK, _D), lambda i: (i, 0))
_acc_spec = pl.BlockSpec((_NC, _ROWBLK, _D), lambda i: (0, i, 0))
_deg_spec = pl.BlockSpec((_NC, _ROWBLK, _D), lambda i: (0, i, 0))
_w_spec = pl.BlockSpec((_D, _D), lambda i: (0, 0))
_b_spec = pl.BlockSpec((1, _D), lambda i: (0, 0))

_mm_first = pl.pallas_call(
    _mm_first_body,
    grid=(_GRID,),
    in_specs=[_row_spec, _w_spec, _deg_spec],
    out_specs=_row_spec,
    out_shape=jax.ShapeDtypeStruct((_N, _D), _DT),
)

_mm_mid = pl.pallas_call(
    _mm_mid_body,
    grid=(_GRID,),
    in_specs=[_acc_spec, _row_spec, _deg_spec, _b_spec, _w_spec],
    out_specs=_row_spec,
    out_shape=jax.ShapeDtypeStruct((_N, _D), _DT),
)

_pool = pl.pallas_call(
    _pool_body,
    grid=(_GRID,),
    in_specs=[
        _acc_spec, _row_spec, _deg_spec, _b_spec,
        pl.BlockSpec((1, 1, _ROWBLK), lambda i: (i, 0, 0)),
        pl.BlockSpec((_D, _DPAD), lambda i: (0, 0)),
        pl.BlockSpec((1, _DPAD), lambda i: (0, 0)),
    ],
    out_specs=pl.BlockSpec((_B, _DPAD), lambda i: (0, 0)),
    out_shape=jax.ShapeDtypeStruct((_B, _DPAD), jnp.float32),
    scratch_shapes=[
        pltpu.VMEM((_B, _D), jnp.float32),
        pltpu.VMEM((_B, _D), jnp.float32),
    ],
)


@jax.jit
def kernel(x, edge_index, batch, W1, b1, W2, b2, W3, b3, Wf, bf):
    src = edge_index[0].astype(jnp.int32).reshape(_NW, _EPT)
    dst = edge_index[1].astype(jnp.int32).reshape(_NW, _EPT)
    pad = _EPAD - _EPT
    srcp = jnp.pad(src, ((0, 0), (0, pad))).reshape(_NW, _NCHUNK, _CH)
    dstp = jnp.pad(dst, ((0, 0), (0, pad)),
                   constant_values=_DUMMY).reshape(_NW, _NCHUNK, _CH)

    ones128 = jnp.ones((_CH, _D), _DT)
    z128 = jnp.zeros((_STRIPE, _D), _DT)

    degp = _sc_deg(dstp, ones128, z128)

    b1r = b1.reshape(1, _D)
    b2r = b2.reshape(1, _D)
    b3r = b3.reshape(1, _D)
    wfp = jnp.zeros((_D, _DPAD), jnp.float32).at[:, :_DOUT].set(Wf)
    bfp = jnp.full((1, _DPAD), -1e30, jnp.float32).at[0, :_DOUT].set(bf)
    batch3 = batch.astype(jnp.int32).reshape(_GRID, 1, _ROWBLK)

    hs1 = _mm_first(x, W1, degp)
    acc1 = _sc_agg(hs1, srcp, dstp, z128)
    hs2 = _mm_mid(acc1, hs1, degp, b1r, W2)
    acc2 = _sc_agg(hs2, srcp, dstp, z128)
    hs3 = _mm_mid(acc2, hs2, degp, b2r, W3)
    acc3 = _sc_agg(hs3, srcp, dstp, z128)

    outp = _pool(acc3, hs3, degp, b3r, batch3, wfp, bfp)
    return outp[:, :_DOUT]


# speedup vs baseline: 2.2264x; 2.2264x over previous
import jax
import jax.numpy as jnp
from jax import lax
from jax.experimental import pallas as pl
from jax.experimental.pallas import tpu as pltpu
from jax.experimental.pallas import tpu_sc as plsc

_N = 10000
_E = 320000
_D = 128
_B = 64
_DOUT = 1317
_DPAD = 1408
_NC, _NS = 2, 16
_NW = _NC * _NS
_EPT = _E // _NW
_CH = 128
_NCHUNK = 80
_EPAD = _NCHUNK * _CH
_NP = 10240
_STRIPE = _NP // _NS
_DUMMY = _N
_ROWBLK = 2000
_GRID = _N // _ROWBLK
_DT = jnp.float32


def _sc_deg_body(dst_hbm, ones_hbm, zeros_hbm, out_hbm, idx_v, ones_v, acc,
                 sem):
    c = lax.axis_index("c")
    s = lax.axis_index("s")
    wid = c * _NS + s
    pltpu.sync_copy(zeros_hbm, acc.at[pl.ds(s * _STRIPE, _STRIPE)])
    pltpu.sync_copy(ones_hbm, ones_v)
    pltpu.sync_copy(dst_hbm.at[wid], idx_v)
    plsc.subcore_barrier()

    def fire(j, carry):
        pltpu.async_copy(ones_v, acc.at[idx_v.at[j]], sem, add=True)
        return carry

    lax.fori_loop(0, _NCHUNK, fire, 0)

    def drain(j, carry):
        pltpu.make_async_copy(ones_hbm, ones_v, sem).wait()
        return carry

    lax.fori_loop(0, _NCHUNK, drain, 0)
    plsc.subcore_barrier()
    pltpu.sync_copy(acc.at[pl.ds(s * _STRIPE, _STRIPE)],
                    out_hbm.at[c, pl.ds(s * _STRIPE, _STRIPE)])


_sc_deg = pl.kernel(
    _sc_deg_body,
    out_type=jax.ShapeDtypeStruct((_NC, _NP, _D), _DT),
    mesh=plsc.VectorSubcoreMesh(core_axis_name="c", subcore_axis_name="s"),
    scratch_types=[
        pltpu.VMEM((_NCHUNK, _CH), jnp.int32),
        pltpu.VMEM((_CH, _D), _DT),
        pltpu.VMEM_SHARED((_NP, _D), _DT),
        pltpu.SemaphoreType.DMA,
    ],
)


def _sc_agg_body(hs_hbm, src_hbm, dst_hbm, zeros_hbm, out_hbm,
                 src_v, dst_v, rows_v, acc, sem):
    c = lax.axis_index("c")
    s = lax.axis_index("s")
    wid = c * _NS + s
    pltpu.sync_copy(zeros_hbm, acc.at[pl.ds(s * _STRIPE, _STRIPE)])
    pltpu.sync_copy(src_hbm.at[wid], src_v)
    pltpu.sync_copy(dst_hbm.at[wid], dst_v)
    plsc.subcore_barrier()

    def body(j, carry):
        pltpu.async_copy(hs_hbm.at[src_v.at[j]], rows_v, sem).wait()
        pltpu.sync_copy(rows_v, acc.at[dst_v.at[j]], add=True)
        return carry

    lax.fori_loop(0, _NCHUNK, body, 0)
    plsc.subcore_barrier()
    pltpu.sync_copy(acc.at[pl.ds(s * _STRIPE, _STRIPE)],
                    out_hbm.at[c, pl.ds(s * _STRIPE, _STRIPE)])


_sc_agg = pl.kernel(
    _sc_agg_body,
    out_type=jax.ShapeDtypeStruct((_NC, _NP, _D), _DT),
    mesh=plsc.VectorSubcoreMesh(core_axis_name="c", subcore_axis_name="s"),
    scratch_types=[
        pltpu.VMEM((_NCHUNK, _CH), jnp.int32),
        pltpu.VMEM((_NCHUNK, _CH), jnp.int32),
        pltpu.VMEM((_CH, _D), _DT),
        pltpu.VMEM_SHARED((_NP, _D), _DT),
        pltpu.SemaphoreType.DMA,
    ],
)


def _dinv(deg_ref):
    d = (deg_ref[0, :, 0:1].astype(jnp.float32) +
         deg_ref[1, :, 0:1].astype(jnp.float32))
    return lax.rsqrt(d + 1.0)


def _mm_first_body(x_ref, w_ref, deg_ref, o_ref):
    h = jnp.dot(x_ref[...], w_ref[...],
                preferred_element_type=jnp.float32) * _dinv(deg_ref)
    o_ref[...] = h.astype(_DT)


def _mm_mid_body(acc_ref, hs_ref, deg_ref, b_ref, w_ref, o_ref):
    dinv = _dinv(deg_ref)
    t = (acc_ref[0].astype(jnp.float32) + acc_ref[1].astype(jnp.float32) +
         hs_ref[...].astype(jnp.float32))
    xl = jnp.maximum(t * dinv + b_ref[...], 0.0)
    o_ref[...] = (jnp.dot(xl, w_ref[...],
                          preferred_element_type=jnp.float32) *
                  dinv).astype(_DT)


def _pool_body(acc_ref, hs_ref, deg_ref, b_ref, batch_ref, wf_ref, bf_ref,
               o_ref, sums, cnts):
    i = pl.program_id(0)
    dinv = _dinv(deg_ref)
    t = (acc_ref[0].astype(jnp.float32) + acc_ref[1].astype(jnp.float32) +
         hs_ref[...].astype(jnp.float32))
    xl = jnp.maximum(t * dinv + b_ref[...], 0.0)
    bids = batch_ref[0, 0, :]
    oh = (bids[None, :] ==
          lax.broadcasted_iota(jnp.int32, (_B, _ROWBLK), 0)).astype(jnp.float32)

    @pl.when(i == 0)
    def _():
        sums[...] = jnp.zeros_like(sums)
        cnts[...] = jnp.zeros_like(cnts)

    sums[...] += jnp.dot(oh, xl, preferred_element_type=jnp.float32)
    cnts[...] += jnp.broadcast_to(jnp.sum(oh, axis=1, keepdims=True),
                                  (_B, _D))

    @pl.when(i == _GRID - 1)
    def _():
        pooled = sums[...] / jnp.maximum(cnts[...], 1.0)
        logits = jnp.dot(pooled, wf_ref[...],
                         preferred_element_type=jnp.float32) + bf_ref[...]
        m = jnp.max(logits, axis=1, keepdims=True)
        lse = jnp.log(jnp.sum(jnp.exp(logits - m), axis=1, keepdims=True))
        o_ref[...] = logits - m - lse


_row_spec = pl.BlockSpec((_ROWBLK, _D), lambda i: (i, 0))
_acc_spec = pl.BlockSpec((_NC, _ROWBLK, _D), lambda i: (0, i, 0))
_deg_spec = pl.BlockSpec((_NC, _ROWBLK, _D), lambda i: (0, i, 0))
_w_spec = pl.BlockSpec((_D, _D), lambda i: (0, 0))
_b_spec = pl.BlockSpec((1, _D), lambda i: (0, 0))

_mm_first = pl.pallas_call(
    _mm_first_body,
    grid=(_GRID,),
    in_specs=[_row_spec, _w_spec, _deg_spec],
    out_specs=_row_spec,
    out_shape=jax.ShapeDtypeStruct((_N, _D), _DT),
)

_mm_mid = pl.pallas_call(
    _mm_mid_body,
    grid=(_GRID,),
    in_specs=[_acc_spec, _row_spec, _deg_spec, _b_spec, _w_spec],
    out_specs=_row_spec,
    out_shape=jax.ShapeDtypeStruct((_N, _D), _DT),
)

_pool = pl.pallas_call(
    _pool_body,
    grid=(_GRID,),
    in_specs=[
        _acc_spec, _row_spec, _deg_spec, _b_spec,
        pl.BlockSpec((1, 1, _ROWBLK), lambda i: (i, 0, 0)),
        pl.BlockSpec((_D, _DPAD), lambda i: (0, 0)),
        pl.BlockSpec((1, _DPAD), lambda i: (0, 0)),
    ],
    out_specs=pl.BlockSpec((_B, _DPAD), lambda i: (0, 0)),
    out_shape=jax.ShapeDtypeStruct((_B, _DPAD), jnp.float32),
    scratch_shapes=[
        pltpu.VMEM((_B, _D), jnp.float32),
        pltpu.VMEM((_B, _D), jnp.float32),
    ],
)


@jax.jit
def kernel(x, edge_index, batch, W1, b1, W2, b2, W3, b3, Wf, bf):
    src = edge_index[0].astype(jnp.int32).reshape(_NW, _EPT)
    dst = edge_index[1].astype(jnp.int32).reshape(_NW, _EPT)
    pad = _EPAD - _EPT
    pad_src = jnp.broadcast_to((jnp.arange(pad, dtype=jnp.int32) * 41) % _N,
                               (_NW, pad))
    pad_dst = jnp.broadcast_to(_N + (jnp.arange(pad, dtype=jnp.int32)
                                     % (_NP - _N)), (_NW, pad))
    srcp = jnp.concatenate([src, pad_src], 1).reshape(_NW, _NCHUNK, _CH)
    dstp = jnp.concatenate([dst, pad_dst], 1).reshape(_NW, _NCHUNK, _CH)

    ones128 = jnp.ones((_CH, _D), _DT)
    z128 = jnp.zeros((_STRIPE, _D), _DT)

    degp = _sc_deg(dstp, ones128, z128)

    b1r = b1.reshape(1, _D)
    b2r = b2.reshape(1, _D)
    b3r = b3.reshape(1, _D)
    wfp = jnp.zeros((_D, _DPAD), jnp.float32).at[:, :_DOUT].set(Wf)
    bfp = jnp.full((1, _DPAD), -1e30, jnp.float32).at[0, :_DOUT].set(bf)
    batch3 = batch.astype(jnp.int32).reshape(_GRID, 1, _ROWBLK)

    hs1 = _mm_first(x, W1, degp)
    acc1 = _sc_agg(hs1, srcp, dstp, z128)
    hs2 = _mm_mid(acc1, hs1, degp, b1r, W2)
    acc2 = _sc_agg(hs2, srcp, dstp, z128)
    hs3 = _mm_mid(acc2, hs2, degp, b2r, W3)
    acc3 = _sc_agg(hs3, srcp, dstp, z128)

    outp = _pool(acc3, hs3, degp, b3r, batch3, wfp, bfp)
    return outp[:, :_DOUT]


# speedup vs baseline: 2.7939x; 1.2549x over previous
import jax
import jax.numpy as jnp
from jax import lax
from jax.experimental import pallas as pl
from jax.experimental.pallas import tpu as pltpu
from jax.experimental.pallas import tpu_sc as plsc

_N = 10000
_E = 320000
_D = 128
_B = 64
_DOUT = 1317
_DPAD = 1408
_NC, _NS = 2, 16
_NW = _NC * _NS
_EPT = _E // _NW
_CH = 128
_NCHUNK = 80
_EPAD = _NCHUNK * _CH
_NP = 10240
_STRIPE = _NP // _NS
_DUMMY = _N
_ROWBLK = 2000
_GRID = _N // _ROWBLK
_DT = jnp.float32


def _sc_deg_body(dst_hbm, ones_hbm, zeros_hbm, out_hbm, idx_v, ones_v, acc,
                 sem):
    c = lax.axis_index("c")
    s = lax.axis_index("s")
    wid = c * _NS + s
    pltpu.sync_copy(zeros_hbm, acc.at[pl.ds(s * _STRIPE, _STRIPE)])
    pltpu.sync_copy(ones_hbm, ones_v)
    pltpu.sync_copy(dst_hbm.at[wid], idx_v)
    plsc.subcore_barrier()

    def fire(j, carry):
        pltpu.async_copy(ones_v, acc.at[idx_v.at[j]], sem, add=True)
        return carry

    lax.fori_loop(0, _NCHUNK, fire, 0)

    def drain(j, carry):
        pltpu.make_async_copy(ones_hbm, ones_v, sem).wait()
        return carry

    lax.fori_loop(0, _NCHUNK, drain, 0)
    plsc.subcore_barrier()
    pltpu.sync_copy(acc.at[pl.ds(s * _STRIPE, _STRIPE)],
                    out_hbm.at[c, pl.ds(s * _STRIPE, _STRIPE)])


_sc_deg = pl.kernel(
    _sc_deg_body,
    out_type=jax.ShapeDtypeStruct((_NC, _NP, _D), _DT),
    mesh=plsc.VectorSubcoreMesh(core_axis_name="c", subcore_axis_name="s"),
    scratch_types=[
        pltpu.VMEM((_NCHUNK, _CH), jnp.int32),
        pltpu.VMEM((_CH, _D), _DT),
        pltpu.VMEM_SHARED((_NP, _D), _DT),
        pltpu.SemaphoreType.DMA,
    ],
)


_GSZ = 8
_NGRP = _NCHUNK // _GSZ


def _sc_agg_body(hs_hbm, src_hbm, dst_hbm, zeros_hbm, out_hbm,
                 src_v, dst_v, buf_a, buf_b, acc, isem, gs_a, gs_b):
    c = lax.axis_index("c")
    s = lax.axis_index("s")
    wid = c * _NS + s
    pltpu.sync_copy(zeros_hbm, acc.at[pl.ds(s * _STRIPE, _STRIPE)])
    pltpu.sync_copy(src_hbm.at[wid, pl.ds(0, _GSZ)], src_v.at[pl.ds(0, _GSZ)])
    pltpu.sync_copy(dst_hbm.at[wid, pl.ds(0, _GSZ)], dst_v.at[pl.ds(0, _GSZ)])
    plsc.subcore_barrier()

    slots = ((buf_a, gs_a), (buf_b, gs_b))

    def wait_chunk(buf, sem):
        pltpu.make_async_copy(hs_hbm.at[pl.ds(0, _CH)], buf, sem).wait()

    def wait_idx():
        pltpu.make_async_copy(src_hbm.at[wid, pl.ds(0, _GSZ)],
                              src_v.at[pl.ds(0, _GSZ)], isem).wait()
        pltpu.make_async_copy(dst_hbm.at[wid, pl.ds(0, _GSZ)],
                              dst_v.at[pl.ds(0, _GSZ)], isem).wait()

    pltpu.async_copy(hs_hbm.at[src_v.at[0]], buf_a, gs_a)

    def group(g, carry):
        p = lax.rem(g, 2)
        base = p * _GSZ
        nbase = _GSZ - base
        has_next = g < _NGRP - 1

        @pl.when(has_next)
        def _():
            pltpu.async_copy(src_hbm.at[wid, pl.ds((g + 1) * _GSZ, _GSZ)],
                             src_v.at[pl.ds(nbase, _GSZ)], isem)
            pltpu.async_copy(dst_hbm.at[wid, pl.ds((g + 1) * _GSZ, _GSZ)],
                             dst_v.at[pl.ds(nbase, _GSZ)], isem)

        for b in range(_GSZ):
            buf, gs = slots[b % 2]
            nbuf, ngs = slots[(b + 1) % 2]
            wait_chunk(buf, gs)
            if b < _GSZ - 1:
                pltpu.async_copy(hs_hbm.at[src_v.at[base + b + 1]], nbuf, ngs)
            else:
                @pl.when(has_next)
                def _():
                    wait_idx()
                    pltpu.async_copy(hs_hbm.at[src_v.at[nbase]], nbuf, ngs)
            pltpu.sync_copy(buf, acc.at[dst_v.at[base + b]], add=True)
        return carry

    lax.fori_loop(0, _NGRP, group, 0)
    plsc.subcore_barrier()
    pltpu.sync_copy(acc.at[pl.ds(s * _STRIPE, _STRIPE)],
                    out_hbm.at[c, pl.ds(s * _STRIPE, _STRIPE)])


_sc_agg = pl.kernel(
    _sc_agg_body,
    out_type=jax.ShapeDtypeStruct((_NC, _NP, _D), _DT),
    mesh=plsc.VectorSubcoreMesh(core_axis_name="c", subcore_axis_name="s"),
    scratch_types=[
        pltpu.VMEM((2 * _GSZ, _CH), jnp.int32),
        pltpu.VMEM((2 * _GSZ, _CH), jnp.int32),
        pltpu.VMEM((_CH, _D), _DT),
        pltpu.VMEM((_CH, _D), _DT),
        pltpu.VMEM_SHARED((_NP, _D), _DT),
        pltpu.SemaphoreType.DMA,
        pltpu.SemaphoreType.DMA,
        pltpu.SemaphoreType.DMA,
    ],
)


def _dinv(deg_ref):
    d = (deg_ref[0, :, 0:1].astype(jnp.float32) +
         deg_ref[1, :, 0:1].astype(jnp.float32))
    return lax.rsqrt(d + 1.0)


def _mm_first_body(x_ref, w_ref, deg_ref, o_ref):
    h = jnp.dot(x_ref[...], w_ref[...],
                preferred_element_type=jnp.float32) * _dinv(deg_ref)
    o_ref[...] = h.astype(_DT)


def _mm_mid_body(acc_ref, hs_ref, deg_ref, b_ref, w_ref, o_ref):
    dinv = _dinv(deg_ref)
    t = (acc_ref[0].astype(jnp.float32) + acc_ref[1].astype(jnp.float32) +
         hs_ref[...].astype(jnp.float32))
    xl = jnp.maximum(t * dinv + b_ref[...], 0.0)
    o_ref[...] = (jnp.dot(xl, w_ref[...],
                          preferred_element_type=jnp.float32) *
                  dinv).astype(_DT)


def _pool_body(acc_ref, hs_ref, deg_ref, b_ref, batch_ref, wf_ref, bf_ref,
               o_ref, sums, cnts):
    i = pl.program_id(0)
    dinv = _dinv(deg_ref)
    t = (acc_ref[0].astype(jnp.float32) + acc_ref[1].astype(jnp.float32) +
         hs_ref[...].astype(jnp.float32))
    xl = jnp.maximum(t * dinv + b_ref[...], 0.0)
    bids = batch_ref[0, 0, :]
    oh = (bids[None, :] ==
          lax.broadcasted_iota(jnp.int32, (_B, _ROWBLK), 0)).astype(jnp.float32)

    @pl.when(i == 0)
    def _():
        sums[...] = jnp.zeros_like(sums)
        cnts[...] = jnp.zeros_like(cnts)

    sums[...] += jnp.dot(oh, xl, preferred_element_type=jnp.float32)
    cnts[...] += jnp.broadcast_to(jnp.sum(oh, axis=1, keepdims=True),
                                  (_B, _D))

    @pl.when(i == _GRID - 1)
    def _():
        pooled = sums[...] / jnp.maximum(cnts[...], 1.0)
        logits = jnp.dot(pooled, wf_ref[...],
                         preferred_element_type=jnp.float32) + bf_ref[...]
        m = jnp.max(logits, axis=1, keepdims=True)
        lse = jnp.log(jnp.sum(jnp.exp(logits - m), axis=1, keepdims=True))
        o_ref[...] = logits - m - lse


_row_spec = pl.BlockSpec((_ROWBLK, _D), lambda i: (i, 0))
_acc_spec = pl.BlockSpec((_NC, _ROWBLK, _D), lambda i: (0, i, 0))
_deg_spec = pl.BlockSpec((_NC, _ROWBLK, _D), lambda i: (0, i, 0))
_w_spec = pl.BlockSpec((_D, _D), lambda i: (0, 0))
_b_spec = pl.BlockSpec((1, _D), lambda i: (0, 0))

_mm_first = pl.pallas_call(
    _mm_first_body,
    grid=(_GRID,),
    in_specs=[_row_spec, _w_spec, _deg_spec],
    out_specs=_row_spec,
    out_shape=jax.ShapeDtypeStruct((_N, _D), _DT),
)

_mm_mid = pl.pallas_call(
    _mm_mid_body,
    grid=(_GRID,),
    in_specs=[_acc_spec, _row_spec, _deg_spec, _b_spec, _w_spec],
    out_specs=_row_spec,
    out_shape=jax.ShapeDtypeStruct((_N, _D), _DT),
)

_pool = pl.pallas_call(
    _pool_body,
    grid=(_GRID,),
    in_specs=[
        _acc_spec, _row_spec, _deg_spec, _b_spec,
        pl.BlockSpec((1, 1, _ROWBLK), lambda i: (i, 0, 0)),
        pl.BlockSpec((_D, _DPAD), lambda i: (0, 0)),
        pl.BlockSpec((1, _DPAD), lambda i: (0, 0)),
    ],
    out_specs=pl.BlockSpec((_B, _DPAD), lambda i: (0, 0)),
    out_shape=jax.ShapeDtypeStruct((_B, _DPAD), jnp.float32),
    scratch_shapes=[
        pltpu.VMEM((_B, _D), jnp.float32),
        pltpu.VMEM((_B, _D), jnp.float32),
    ],
)


@jax.jit
def kernel(x, edge_index, batch, W1, b1, W2, b2, W3, b3, Wf, bf):
    src = edge_index[0].astype(jnp.int32).reshape(_NW, _EPT)
    dst = edge_index[1].astype(jnp.int32).reshape(_NW, _EPT)
    pad = _EPAD - _EPT
    pad_src = jnp.broadcast_to((jnp.arange(pad, dtype=jnp.int32) * 41) % _N,
                               (_NW, pad))
    pad_dst = jnp.broadcast_to(_N + (jnp.arange(pad, dtype=jnp.int32)
                                     % (_NP - _N)), (_NW, pad))
    srcp = jnp.concatenate([src, pad_src], 1).reshape(_NW, _NCHUNK, _CH)
    dstp = jnp.concatenate([dst, pad_dst], 1).reshape(_NW, _NCHUNK, _CH)

    ones128 = jnp.ones((_CH, _D), _DT)
    z128 = jnp.zeros((_STRIPE, _D), _DT)

    degp = _sc_deg(dstp, ones128, z128)

    b1r = b1.reshape(1, _D)
    b2r = b2.reshape(1, _D)
    b3r = b3.reshape(1, _D)
    wfp = jnp.zeros((_D, _DPAD), jnp.float32).at[:, :_DOUT].set(Wf)
    bfp = jnp.full((1, _DPAD), -1e30, jnp.float32).at[0, :_DOUT].set(bf)
    batch3 = batch.astype(jnp.int32).reshape(_GRID, 1, _ROWBLK)

    hs1 = _mm_first(x, W1, degp)
    acc1 = _sc_agg(hs1, srcp, dstp, z128)
    hs2 = _mm_mid(acc1, hs1, degp, b1r, W2)
    acc2 = _sc_agg(hs2, srcp, dstp, z128)
    hs3 = _mm_mid(acc2, hs2, degp, b2r, W3)
    acc3 = _sc_agg(hs3, srcp, dstp, z128)

    outp = _pool(acc3, hs3, degp, b3r, batch3, wfp, bfp)
    return outp[:, :_DOUT]


# speedup vs baseline: 2.8012x; 1.0026x over previous
import jax
import jax.numpy as jnp
from jax import lax
from jax.experimental import pallas as pl
from jax.experimental.pallas import tpu as pltpu
from jax.experimental.pallas import tpu_sc as plsc

_N = 10000
_E = 320000
_D = 128
_B = 64
_DOUT = 1317
_DPAD = 1408
_NC, _NS = 2, 16
_NW = _NC * _NS
_EPT = _E // _NW
_CH = 128
_NCHUNK = 80
_EPAD = _NCHUNK * _CH
_NP = 10240
_STRIPE = _NP // _NS
_ROWBLK = 2000
_GRID = _N // _ROWBLK
_DT = jnp.float32


def _sc_deg_body(dst_hbm, ones_hbm, zeros_hbm, out_hbm, idx_v, ones_v, acc,
                 sem):
    c = lax.axis_index("c")
    s = lax.axis_index("s")
    wid = c * _NS + s
    pltpu.sync_copy(zeros_hbm, acc.at[pl.ds(s * _STRIPE, _STRIPE)])
    pltpu.sync_copy(ones_hbm, ones_v)
    pltpu.sync_copy(dst_hbm.at[wid], idx_v)
    plsc.subcore_barrier()

    def fire(j, carry):
        pltpu.async_copy(ones_v, acc.at[idx_v.at[j]], sem, add=True)
        return carry

    lax.fori_loop(0, _NCHUNK, fire, 0)

    def drain(j, carry):
        pltpu.make_async_copy(ones_hbm, ones_v, sem).wait()
        return carry

    lax.fori_loop(0, _NCHUNK, drain, 0)
    plsc.subcore_barrier()
    pltpu.sync_copy(acc.at[pl.ds(s * _STRIPE, _STRIPE)],
                    out_hbm.at[c, pl.ds(s * _STRIPE, _STRIPE)])


_sc_deg = pl.kernel(
    _sc_deg_body,
    out_type=jax.ShapeDtypeStruct((_NC, _NP, _D), _DT),
    mesh=plsc.VectorSubcoreMesh(core_axis_name="c", subcore_axis_name="s"),
    scratch_types=[
        pltpu.VMEM((_NCHUNK, _CH), jnp.int32),
        pltpu.VMEM((_CH, _D), _DT),
        pltpu.VMEM_SHARED((_NP, _D), _DT),
        pltpu.SemaphoreType.DMA,
    ],
)


_GSZ = 8
_NGRP = _NCHUNK // _GSZ


def _sc_agg_body(hs_hbm, src_hbm, dst_hbm, zeros_hbm, out_hbm,
                 src_v, dst_v, buf_a, buf_b, acc, isem, gs_a, gs_b):
    c = lax.axis_index("c")
    s = lax.axis_index("s")
    wid = c * _NS + s
    pltpu.sync_copy(zeros_hbm, acc.at[pl.ds(s * _STRIPE, _STRIPE)])
    pltpu.sync_copy(src_hbm.at[wid, pl.ds(0, _GSZ)], src_v.at[pl.ds(0, _GSZ)])
    pltpu.sync_copy(dst_hbm.at[wid, pl.ds(0, _GSZ)], dst_v.at[pl.ds(0, _GSZ)])
    plsc.subcore_barrier()

    slots = ((buf_a, gs_a), (buf_b, gs_b))

    def wait_chunk(buf, sem):
        pltpu.make_async_copy(hs_hbm.at[pl.ds(0, _CH)], buf, sem).wait()

    def wait_idx():
        pltpu.make_async_copy(src_hbm.at[wid, pl.ds(0, _GSZ)],
                              src_v.at[pl.ds(0, _GSZ)], isem).wait()
        pltpu.make_async_copy(dst_hbm.at[wid, pl.ds(0, _GSZ)],
                              dst_v.at[pl.ds(0, _GSZ)], isem).wait()

    pltpu.async_copy(hs_hbm.at[src_v.at[0]], buf_a, gs_a)

    def group(g, carry):
        p = lax.rem(g, 2)
        base = p * _GSZ
        nbase = _GSZ - base
        has_next = g < _NGRP - 1

        @pl.when(has_next)
        def _():
            pltpu.async_copy(src_hbm.at[wid, pl.ds((g + 1) * _GSZ, _GSZ)],
                             src_v.at[pl.ds(nbase, _GSZ)], isem)
            pltpu.async_copy(dst_hbm.at[wid, pl.ds((g + 1) * _GSZ, _GSZ)],
                             dst_v.at[pl.ds(nbase, _GSZ)], isem)

        for b in range(_GSZ):
            buf, gs = slots[b % 2]
            nbuf, ngs = slots[(b + 1) % 2]
            wait_chunk(buf, gs)
            if b < _GSZ - 1:
                pltpu.async_copy(hs_hbm.at[src_v.at[base + b + 1]], nbuf, ngs)
            else:
                @pl.when(has_next)
                def _():
                    wait_idx()
                    pltpu.async_copy(hs_hbm.at[src_v.at[nbase]], nbuf, ngs)
            pltpu.sync_copy(buf, acc.at[dst_v.at[base + b]], add=True)
        return carry

    lax.fori_loop(0, _NGRP, group, 0)
    plsc.subcore_barrier()
    pltpu.sync_copy(acc.at[pl.ds(s * _STRIPE, _STRIPE)],
                    out_hbm.at[c, pl.ds(s * _STRIPE, _STRIPE)])


_sc_agg = pl.kernel(
    _sc_agg_body,
    out_type=jax.ShapeDtypeStruct((_NC, _NP, _D), _DT),
    mesh=plsc.VectorSubcoreMesh(core_axis_name="c", subcore_axis_name="s"),
    scratch_types=[
        pltpu.VMEM((2 * _GSZ, _CH), jnp.int32),
        pltpu.VMEM((2 * _GSZ, _CH), jnp.int32),
        pltpu.VMEM((_CH, _D), _DT),
        pltpu.VMEM((_CH, _D), _DT),
        pltpu.VMEM_SHARED((_NP, _D), _DT),
        pltpu.SemaphoreType.DMA,
        pltpu.SemaphoreType.DMA,
        pltpu.SemaphoreType.DMA,
    ],
)


def _dinv(deg_ref):
    d = (deg_ref[0, :, 0:1].astype(jnp.float32) +
         deg_ref[1, :, 0:1].astype(jnp.float32))
    return lax.rsqrt(d + 1.0)


def _mm_first_body(x_ref, w_ref, deg_ref, o_ref):
    h = jnp.dot(x_ref[...], w_ref[...],
                preferred_element_type=jnp.float32) * _dinv(deg_ref)
    o_ref[...] = h.astype(_DT)


def _mm_mid_body(acc_ref, hs_ref, deg_ref, b_ref, w_ref, o_ref):
    dinv = _dinv(deg_ref)
    t = (acc_ref[0].astype(jnp.float32) + acc_ref[1].astype(jnp.float32) +
         hs_ref[...].astype(jnp.float32))
    xl = jnp.maximum(t * dinv + b_ref[...], 0.0)
    o_ref[...] = (jnp.dot(xl, w_ref[...],
                          preferred_element_type=jnp.float32) *
                  dinv).astype(_DT)


def _pool_body(acc_ref, hs_ref, deg_ref, b_ref, batch_ref, wf_ref, bf_ref,
               o_ref, sums, cnts):
    i = pl.program_id(0)
    dinv = _dinv(deg_ref)
    t = (acc_ref[0].astype(jnp.float32) + acc_ref[1].astype(jnp.float32) +
         hs_ref[...].astype(jnp.float32))
    xl = jnp.maximum(t * dinv + b_ref[...], 0.0)
    bids = batch_ref[0, 0, :]
    oh = (bids[None, :] ==
          lax.broadcasted_iota(jnp.int32, (_B, _ROWBLK), 0)).astype(jnp.float32)

    @pl.when(i == 0)
    def _():
        sums[...] = jnp.zeros_like(sums)
        cnts[...] = jnp.zeros_like(cnts)

    sums[...] += jnp.dot(oh, xl, preferred_element_type=jnp.float32)
    cnts[...] += jnp.broadcast_to(jnp.sum(oh, axis=1, keepdims=True),
                                  (_B, _D))

    @pl.when(i == _GRID - 1)
    def _():
        pooled = sums[...] / jnp.maximum(cnts[...], 1.0)
        logits = jnp.dot(pooled, wf_ref[...],
                         preferred_element_type=jnp.float32) + bf_ref[...]
        m = jnp.max(logits, axis=1, keepdims=True)
        lse = jnp.log(jnp.sum(jnp.exp(logits - m), axis=1, keepdims=True))
        o_ref[...] = logits - m - lse


_row_spec = pl.BlockSpec((_ROWBLK, _D), lambda i: (i, 0))
_acc_spec = pl.BlockSpec((_NC, _ROWBLK, _D), lambda i: (0, i, 0))
_deg_spec = pl.BlockSpec((_NC, _ROWBLK, _D), lambda i: (0, i, 0))
_w_spec = pl.BlockSpec((_D, _D), lambda i: (0, 0))
_b_spec = pl.BlockSpec((1, _D), lambda i: (0, 0))

_mm_first = pl.pallas_call(
    _mm_first_body,
    grid=(_GRID,),
    in_specs=[_row_spec, _w_spec, _deg_spec],
    out_specs=_row_spec,
    out_shape=jax.ShapeDtypeStruct((_N, _D), _DT),
)

_mm_mid = pl.pallas_call(
    _mm_mid_body,
    grid=(_GRID,),
    in_specs=[_acc_spec, _row_spec, _deg_spec, _b_spec, _w_spec],
    out_specs=_row_spec,
    out_shape=jax.ShapeDtypeStruct((_N, _D), _DT),
)

_pool = pl.pallas_call(
    _pool_body,
    grid=(_GRID,),
    in_specs=[
        _acc_spec, _row_spec, _deg_spec, _b_spec,
        pl.BlockSpec((1, 1, _ROWBLK), lambda i: (i, 0, 0)),
        pl.BlockSpec((_D, _DPAD), lambda i: (0, 0)),
        pl.BlockSpec((1, _DPAD), lambda i: (0, 0)),
    ],
    out_specs=pl.BlockSpec((_B, _DPAD), lambda i: (0, 0)),
    out_shape=jax.ShapeDtypeStruct((_B, _DPAD), jnp.float32),
    scratch_shapes=[
        pltpu.VMEM((_B, _D), jnp.float32),
        pltpu.VMEM((_B, _D), jnp.float32),
    ],
)


@jax.jit
def kernel(x, edge_index, batch, W1, b1, W2, b2, W3, b3, Wf, bf):
    src = edge_index[0].astype(jnp.int32).reshape(_NW, _EPT)
    dst = edge_index[1].astype(jnp.int32).reshape(_NW, _EPT)
    pad = _EPAD - _EPT
    pad_src = jnp.broadcast_to((jnp.arange(pad, dtype=jnp.int32) * 41) % _N,
                               (_NW, pad))
    pad_dst = jnp.broadcast_to(_N + (jnp.arange(pad, dtype=jnp.int32)
                                     % (_NP - _N)), (_NW, pad))
    srcp = jnp.concatenate([src, pad_src], 1).reshape(_NW, _NCHUNK, _CH)
    dstp = jnp.concatenate([dst, pad_dst], 1).reshape(_NW, _NCHUNK, _CH)

    ones128 = jnp.ones((_CH, _D), _DT)
    z128 = jnp.zeros((_STRIPE, _D), _DT)

    degp = _sc_deg(dstp, ones128, z128)

    b1r = b1.reshape(1, _D)
    b2r = b2.reshape(1, _D)
    b3r = b3.reshape(1, _D)
    wfp = jnp.zeros((_D, _DPAD), jnp.float32).at[:, :_DOUT].set(Wf)
    bfp = jnp.full((1, _DPAD), -1e30, jnp.float32).at[0, :_DOUT].set(bf)
    batch3 = batch.astype(jnp.int32).reshape(_GRID, 1, _ROWBLK)

    hs1 = _mm_first(x, W1, degp)
    acc1 = _sc_agg(hs1, srcp, dstp, z128)
    hs2 = _mm_mid(acc1, hs1, degp, b1r, W2)
    acc2 = _sc_agg(hs2, srcp, dstp, z128)
    hs3 = _mm_mid(acc2, hs2, degp, b2r, W3)
    acc3 = _sc_agg(hs3, srcp, dstp, z128)

    outp = _pool(acc3, hs3, degp, b3r, batch3, wfp, bfp)
    return outp[:, :_DOUT]


# speedup vs baseline: 3.4212x; 1.2213x over previous
import jax
import jax.numpy as jnp
from jax import lax
from jax.experimental import pallas as pl
from jax.experimental.pallas import tpu as pltpu
from jax.experimental.pallas import tpu_sc as plsc

_N = 10000
_E = 320000
_D = 128
_B = 64
_DOUT = 1317
_DPAD = 1408
_NC, _NS = 2, 16
_NW = _NC * _NS
_EPT = _E // _NW
_CH = 80
_NCHUNK = 128
_EPAD = _NCHUNK * _CH
_NP = 10240
_STRIPE = _NP // _NS
_ROWBLK = 2000
_GRID = _N // _ROWBLK
_DT = jnp.float32


def _sc_deg_body(dst_hbm, ones_hbm, zeros_hbm, out_hbm, idx_v, ones_v, acc,
                 sem):
    c = lax.axis_index("c")
    s = lax.axis_index("s")
    wid = c * _NS + s
    pltpu.sync_copy(zeros_hbm, acc.at[pl.ds(s * _STRIPE, _STRIPE)])
    pltpu.sync_copy(ones_hbm, ones_v)
    pltpu.sync_copy(dst_hbm.at[wid], idx_v)
    plsc.subcore_barrier()

    def fire(j, carry):
        pltpu.async_copy(ones_v, acc.at[idx_v.at[j]], sem, add=True)
        return carry

    lax.fori_loop(0, _NCHUNK, fire, 0)

    def drain(j, carry):
        pltpu.make_async_copy(ones_hbm, ones_v, sem).wait()
        return carry

    lax.fori_loop(0, _NCHUNK, drain, 0)
    plsc.subcore_barrier()
    pltpu.sync_copy(acc.at[pl.ds(s * _STRIPE, _STRIPE)],
                    out_hbm.at[c, pl.ds(s * _STRIPE, _STRIPE)])


_sc_deg = pl.kernel(
    _sc_deg_body,
    out_type=jax.ShapeDtypeStruct((_NC, _NP, _D), _DT),
    mesh=plsc.VectorSubcoreMesh(core_axis_name="c", subcore_axis_name="s"),
    scratch_types=[
        pltpu.VMEM((_NCHUNK, _CH), jnp.int32),
        pltpu.VMEM((_CH, _D), _DT),
        pltpu.VMEM_SHARED((_NP, _D), _DT),
        pltpu.SemaphoreType.DMA,
    ],
)


_GSZ = 8
_NGRP = _NCHUNK // _GSZ


def _sc_agg_body(hs_hbm, src_hbm, dst_hbm, zeros_hbm, out_hbm,
                 src_v, dst_v, buf_a, buf_b, buf_c, buf_d, acc,
                 isem, gs_a, gs_b, gs_c, gs_d):
    c = lax.axis_index("c")
    s = lax.axis_index("s")
    wid = c * _NS + s
    pltpu.sync_copy(zeros_hbm, acc.at[pl.ds(s * _STRIPE, _STRIPE)])
    pltpu.sync_copy(src_hbm.at[wid, pl.ds(0, _GSZ)], src_v.at[pl.ds(0, _GSZ)])
    pltpu.sync_copy(dst_hbm.at[wid, pl.ds(0, _GSZ)], dst_v.at[pl.ds(0, _GSZ)])
    plsc.subcore_barrier()

    slots = ((buf_a, gs_a), (buf_b, gs_b), (buf_c, gs_c), (buf_d, gs_d))

    def wait_chunk(buf, sem):
        pltpu.make_async_copy(hs_hbm.at[pl.ds(0, _CH)], buf, sem).wait()

    def wait_idx():
        pltpu.make_async_copy(src_hbm.at[wid, pl.ds(0, _GSZ)],
                              src_v.at[pl.ds(0, _GSZ)], isem).wait()
        pltpu.make_async_copy(dst_hbm.at[wid, pl.ds(0, _GSZ)],
                              dst_v.at[pl.ds(0, _GSZ)], isem).wait()

    pltpu.async_copy(hs_hbm.at[src_v.at[0]], buf_a, gs_a)
    pltpu.async_copy(hs_hbm.at[src_v.at[1]], buf_b, gs_b)
    pltpu.async_copy(hs_hbm.at[src_v.at[2]], buf_c, gs_c)

    def group(g, carry):
        p = lax.rem(g, 2)
        base = p * _GSZ
        nbase = _GSZ - base
        has_next = g < _NGRP - 1

        @pl.when(has_next)
        def _():
            pltpu.async_copy(src_hbm.at[wid, pl.ds((g + 1) * _GSZ, _GSZ)],
                             src_v.at[pl.ds(nbase, _GSZ)], isem)
            pltpu.async_copy(dst_hbm.at[wid, pl.ds((g + 1) * _GSZ, _GSZ)],
                             dst_v.at[pl.ds(nbase, _GSZ)], isem)

        for b in range(_GSZ):
            buf, gs = slots[b % 4]
            nbuf, ngs = slots[(b + 3) % 4]
            wait_chunk(buf, gs)
            if b < _GSZ - 3:
                pltpu.async_copy(hs_hbm.at[src_v.at[base + b + 3]], nbuf, ngs)
            elif b == _GSZ - 3:
                @pl.when(has_next)
                def _():
                    wait_idx()
                    pltpu.async_copy(hs_hbm.at[src_v.at[nbase]], nbuf, ngs)
            else:
                nxt = nbase + b - (_GSZ - 3)

                @pl.when(has_next)
                def _():
                    pltpu.async_copy(hs_hbm.at[src_v.at[nxt]], nbuf, ngs)
            pltpu.sync_copy(buf, acc.at[dst_v.at[base + b]], add=True)
        return carry

    lax.fori_loop(0, _NGRP, group, 0)
    plsc.subcore_barrier()
    pltpu.sync_copy(acc.at[pl.ds(s * _STRIPE, _STRIPE)],
                    out_hbm.at[c, pl.ds(s * _STRIPE, _STRIPE)])


_sc_agg = pl.kernel(
    _sc_agg_body,
    out_type=jax.ShapeDtypeStruct((_NC, _NP, _D), _DT),
    mesh=plsc.VectorSubcoreMesh(core_axis_name="c", subcore_axis_name="s"),
    scratch_types=[
        pltpu.VMEM((2 * _GSZ, _CH), jnp.int32),
        pltpu.VMEM((2 * _GSZ, _CH), jnp.int32),
        pltpu.VMEM((_CH, _D), _DT),
        pltpu.VMEM((_CH, _D), _DT),
        pltpu.VMEM((_CH, _D), _DT),
        pltpu.VMEM((_CH, _D), _DT),
        pltpu.VMEM_SHARED((_NP, _D), _DT),
        pltpu.SemaphoreType.DMA,
        pltpu.SemaphoreType.DMA,
        pltpu.SemaphoreType.DMA,
        pltpu.SemaphoreType.DMA,
        pltpu.SemaphoreType.DMA,
    ],
)


def _dinv(deg_ref):
    d = (deg_ref[0, :, 0:1].astype(jnp.float32) +
         deg_ref[1, :, 0:1].astype(jnp.float32))
    return lax.rsqrt(d + 1.0)


def _mm_first_body(x_ref, w_ref, deg_ref, o_ref):
    h = jnp.dot(x_ref[...], w_ref[...],
                preferred_element_type=jnp.float32) * _dinv(deg_ref)
    o_ref[...] = h.astype(_DT)


def _mm_mid_body(acc_ref, hs_ref, deg_ref, b_ref, w_ref, o_ref):
    dinv = _dinv(deg_ref)
    t = (acc_ref[0].astype(jnp.float32) + acc_ref[1].astype(jnp.float32) +
         hs_ref[...].astype(jnp.float32))
    xl = jnp.maximum(t * dinv + b_ref[...], 0.0)
    o_ref[...] = (jnp.dot(xl, w_ref[...],
                          preferred_element_type=jnp.float32) *
                  dinv).astype(_DT)


def _pool_body(acc_ref, hs_ref, deg_ref, b_ref, batch_ref, wf_ref, bf_ref,
               o_ref, sums, cnts):
    i = pl.program_id(0)
    dinv = _dinv(deg_ref)
    t = (acc_ref[0].astype(jnp.float32) + acc_ref[1].astype(jnp.float32) +
         hs_ref[...].astype(jnp.float32))
    xl = jnp.maximum(t * dinv + b_ref[...], 0.0)
    bids = batch_ref[0, 0, :]
    oh = (bids[None, :] ==
          lax.broadcasted_iota(jnp.int32, (_B, _ROWBLK), 0)).astype(jnp.float32)

    @pl.when(i == 0)
    def _():
        sums[...] = jnp.zeros_like(sums)
        cnts[...] = jnp.zeros_like(cnts)

    sums[...] += jnp.dot(oh, xl, preferred_element_type=jnp.float32)
    cnts[...] += jnp.broadcast_to(jnp.sum(oh, axis=1, keepdims=True),
                                  (_B, _D))

    @pl.when(i == _GRID - 1)
    def _():
        pooled = sums[...] / jnp.maximum(cnts[...], 1.0)
        logits = jnp.dot(pooled, wf_ref[...],
                         preferred_element_type=jnp.float32) + bf_ref[...]
        m = jnp.max(logits, axis=1, keepdims=True)
        lse = jnp.log(jnp.sum(jnp.exp(logits - m), axis=1, keepdims=True))
        o_ref[...] = logits - m - lse


_row_spec = pl.BlockSpec((_ROWBLK, _D), lambda i: (i, 0))
_acc_spec = pl.BlockSpec((_NC, _ROWBLK, _D), lambda i: (0, i, 0))
_deg_spec = pl.BlockSpec((_NC, _ROWBLK, _D), lambda i: (0, i, 0))
_w_spec = pl.BlockSpec((_D, _D), lambda i: (0, 0))
_b_spec = pl.BlockSpec((1, _D), lambda i: (0, 0))

_mm_first = pl.pallas_call(
    _mm_first_body,
    grid=(_GRID,),
    in_specs=[_row_spec, _w_spec, _deg_spec],
    out_specs=_row_spec,
    out_shape=jax.ShapeDtypeStruct((_N, _D), _DT),
)

_mm_mid = pl.pallas_call(
    _mm_mid_body,
    grid=(_GRID,),
    in_specs=[_acc_spec, _row_spec, _deg_spec, _b_spec, _w_spec],
    out_specs=_row_spec,
    out_shape=jax.ShapeDtypeStruct((_N, _D), _DT),
)

_pool = pl.pallas_call(
    _pool_body,
    grid=(_GRID,),
    in_specs=[
        _acc_spec, _row_spec, _deg_spec, _b_spec,
        pl.BlockSpec((1, 1, _ROWBLK), lambda i: (i, 0, 0)),
        pl.BlockSpec((_D, _DPAD), lambda i: (0, 0)),
        pl.BlockSpec((1, _DPAD), lambda i: (0, 0)),
    ],
    out_specs=pl.BlockSpec((_B, _DPAD), lambda i: (0, 0)),
    out_shape=jax.ShapeDtypeStruct((_B, _DPAD), jnp.float32),
    scratch_shapes=[
        pltpu.VMEM((_B, _D), jnp.float32),
        pltpu.VMEM((_B, _D), jnp.float32),
    ],
)


@jax.jit
def kernel(x, edge_index, batch, W1, b1, W2, b2, W3, b3, Wf, bf):
    src = edge_index[0].astype(jnp.int32).reshape(_NW, _EPT)
    dst = edge_index[1].astype(jnp.int32).reshape(_NW, _EPT)
    pad = _EPAD - _EPT
    pad_src = jnp.broadcast_to((jnp.arange(pad, dtype=jnp.int32) * 41) % _N,
                               (_NW, pad))
    pad_dst = jnp.broadcast_to(_N + (jnp.arange(pad, dtype=jnp.int32)
                                     % (_NP - _N)), (_NW, pad))
    srcp = jnp.concatenate([src, pad_src], 1).reshape(_NW, _NCHUNK, _CH)
    dstp = jnp.concatenate([dst, pad_dst], 1).reshape(_NW, _NCHUNK, _CH)

    ones128 = jnp.ones((_CH, _D), _DT)
    z128 = jnp.zeros((_STRIPE, _D), _DT)

    degp = _sc_deg(dstp, ones128, z128)

    b1r = b1.reshape(1, _D)
    b2r = b2.reshape(1, _D)
    b3r = b3.reshape(1, _D)
    wfp = jnp.zeros((_D, _DPAD), jnp.float32).at[:, :_DOUT].set(Wf)
    bfp = jnp.full((1, _DPAD), -1e30, jnp.float32).at[0, :_DOUT].set(bf)
    batch3 = batch.astype(jnp.int32).reshape(_GRID, 1, _ROWBLK)

    hs1 = _mm_first(x, W1, degp)
    acc1 = _sc_agg(hs1, srcp, dstp, z128)
    hs2 = _mm_mid(acc1, hs1, degp, b1r, W2)
    acc2 = _sc_agg(hs2, srcp, dstp, z128)
    hs3 = _mm_mid(acc2, hs2, degp, b2r, W3)
    acc3 = _sc_agg(hs3, srcp, dstp, z128)

    outp = _pool(acc3, hs3, degp, b3r, batch3, wfp, bfp)
    return outp[:, :_DOUT]
